# baseline (speedup 1.0000x reference)
import functools

import jax
import jax.numpy as jnp
from jax import lax
from jax.experimental import pallas as pl
from jax.experimental.pallas import tpu as pltpu
from jax.experimental.pallas import tpu_sc as plsc

N_NODES = 10000
N_EDGES = 320000
D_FEAT = 128
N_HEADS = 3
N_CH = 12
ROW_W = 48
K_EDGES = 80
N_GRP = K_EDGES // 16
N_TILES = 16
N_WORKERS = 32
N_CHUNKS_TOT = N_EDGES // K_EDGES
CHUNKS_PER_W = N_CHUNKS_TOT // N_WORKERS
NBUF = 5
N_MACRO = CHUNKS_PER_W // NBUF
LAST = CHUNKS_PER_W - 1
ROWS_PER_TILE = 632


def _prep_body(xs_ref, xt_ref, ea_ref, wcat_ref, vdst_ref, vedge_ref,
               t_ref, adst_ref, ae_ref):
    t_ref[...] = jnp.dot(xs_ref[...], wcat_ref[...],
                         preferred_element_type=jnp.float32)
    adst_ref[...] = jnp.dot(xt_ref[...], vdst_ref[...],
                            preferred_element_type=jnp.float32)
    ae_ref[...] = jnp.dot(ea_ref[...], vedge_ref[...],
                          preferred_element_type=jnp.float32)


def _sc_edge_body(t_hbm, adst_hbm, sdp_hbm, ae_hbm, part_hbm, *scr):
    sd = scr[0:5]
    aer = scr[5:10]
    rows = scr[10:15]
    adrow = scr[15:20]
    dscat = scr[20:25]
    acc = scr[25]
    sem_i = scr[26:31]
    sem_g = scr[31:36]
    sem_s = scr[36:41]

    core = lax.axis_index("c")
    s = lax.axis_index("s")
    w = s * 2 + core

    iota16 = lax.iota(jnp.int32, 16)
    zero16 = jnp.zeros((16,), jnp.float32)

    def col_idx(col):
        return jnp.full((16,), col, jnp.int32)

    for r in range(K_EDGES):
        for o3 in range(ROW_W // 16):
            rows[0][r, pl.ds(o3 * 16, 16)] = zero16
    row_off = pl.multiple_of(
        jnp.minimum(s * ROWS_PER_TILE, N_NODES - ROWS_PER_TILE), 8)
    n_z = ROWS_PER_TILE // K_EDGES + 1
    for k in range(n_z):
        off_k = min(k * K_EDGES, ROWS_PER_TILE - K_EDGES)
        pltpu.sync_copy(rows[0],
                        acc.at[pl.ds(row_off + off_k, K_EDGES)])

    def issue_loads(c, b):
        chn = c * N_WORKERS + w
        pltpu.async_copy(sdp_hbm.at[chn], sd[b], sem_i[b])
        pltpu.async_copy(ae_hbm.at[pl.ds(chn * K_EDGES, K_EDGES)],
                         aer[b], sem_i[b])

    def wait_loads(c, b):
        chn = c * N_WORKERS + w
        pltpu.make_async_copy(sdp_hbm.at[chn], sd[b], sem_i[b]).wait()
        pltpu.make_async_copy(ae_hbm.at[pl.ds(chn * K_EDGES, K_EDGES)],
                              aer[b], sem_i[b]).wait()

    def issue_gather(b):
        pltpu.async_copy(t_hbm.at[sd[b].at[0]], rows[b], sem_g[b])
        pltpu.async_copy(adst_hbm.at[sd[b].at[1]], adrow[b], sem_g[b])

    def wait_gather(b):
        pltpu.make_async_copy(t_hbm.at[sd[b].at[0]], rows[b],
                              sem_g[b]).wait()
        pltpu.make_async_copy(adst_hbm.at[sd[b].at[1]], adrow[b],
                              sem_g[b]).wait()

    def issue_scatter(b):
        pltpu.async_copy(rows[b], acc.at[dscat[b]], sem_s[b], add=True)

    def wait_scatter(b):
        pltpu.make_async_copy(rows[b], acc.at[dscat[b]], sem_s[b]).wait()

    def compute(b):
        for g in range(N_GRP):
            r16 = iota16 + g * 16
            dst16 = sd[b][1, pl.ds(g * 16, 16)]
            dscat[b][pl.ds(g * 16, 16)] = dst16
            for h in range(N_HEADS):
                a_src = plsc.load_gather(rows[b],
                                         [r16, col_idx(16 * h + 12)])
                a_dst = plsc.load_gather(adrow[b], [r16, col_idx(h)])
                a_e = plsc.load_gather(aer[b], [r16, col_idx(h)])
                al = a_src + a_dst + a_e
                al = jnp.maximum(al, al * 0.2)
                ex = jnp.exp(al)
                plsc.store_scatter(rows[b], [r16, col_idx(16 * h + 12)], ex)
                for cc in range(N_CH):
                    col = 16 * h + cc
                    v = plsc.load_gather(rows[b], [r16, col_idx(col)])
                    plsc.store_scatter(rows[b], [r16, col_idx(col)], v * ex)

    plsc.subcore_barrier()

    for c in range(4):
        issue_loads(c, c)
    wait_loads(0, 0)
    wait_loads(1, 1)
    issue_gather(0)
    issue_gather(1)

    def macro_body(m, carry):
        for b in range(NBUF):
            c = m * NBUF + b
            wait_gather(b)

            @pl.when((c >= 3) & (c + 2 <= LAST))
            def _():
                wait_scatter((b + 2) % NBUF)

            @pl.when(c + 2 <= LAST)
            def _():
                wait_loads(c + 2, (b + 2) % NBUF)
                issue_gather((b + 2) % NBUF)

            @pl.when(c + 4 <= LAST)
            def _():
                issue_loads(c + 4, (b + 4) % NBUF)

            compute(b)
            issue_scatter(b)
        return carry

    lax.fori_loop(0, N_MACRO, macro_body, 0)
    for b in range(NBUF):
        wait_scatter(b)

    plsc.subcore_barrier()
    pltpu.sync_copy(acc.at[pl.ds(row_off, ROWS_PER_TILE)],
                    part_hbm.at[core, pl.ds(row_off, ROWS_PER_TILE)])


_sc_edge = functools.partial(
    pl.kernel,
    out_type=jax.ShapeDtypeStruct((2, N_NODES, ROW_W), jnp.float32),
    mesh=plsc.VectorSubcoreMesh(core_axis_name="c", subcore_axis_name="s"),
    compiler_params=pltpu.CompilerParams(needs_layout_passes=False,
                                         use_tc_tiling_on_sc=False),
    scratch_types=(
        [pltpu.VMEM((2, K_EDGES), jnp.int32) for _ in range(NBUF)]
        + [pltpu.VMEM((K_EDGES, 4), jnp.float32) for _ in range(NBUF)]
        + [pltpu.VMEM((K_EDGES, ROW_W), jnp.float32) for _ in range(NBUF)]
        + [pltpu.VMEM((K_EDGES, 8), jnp.float32) for _ in range(NBUF)]
        + [pltpu.VMEM((K_EDGES,), jnp.int32) for _ in range(NBUF)]
        + [pltpu.VMEM_SHARED((N_NODES, ROW_W), jnp.float32)]
        + [pltpu.SemaphoreType.DMA for _ in range(3 * NBUF)]
    ),
)(_sc_edge_body)


def _final_body(part_ref, batch_ref, bpad_ref, w1_ref, b1_ref,
                w3_ref, b3_ref, y_ref):
    p = part_ref[0] + part_ref[1]
    i2 = lax.broadcasted_iota(jnp.int32, (ROW_W, ROW_W), 0)
    j2 = lax.broadcasted_iota(jnp.int32, (ROW_W, ROW_W), 1)
    sel = jnp.where((i2 == (j2 // 16) * 16 + 12) & (j2 % 16 < 12), 1.0, 0.0)
    den = jnp.dot(p, sel, preferred_element_type=jnp.float32)
    x = jnp.maximum(p / (den + 1e-16) + bpad_ref[...], 0.0)
    bi = lax.broadcasted_iota(jnp.int32, (64, N_NODES), 0)
    oht = jnp.where(bi == batch_ref[...], 1.0, 0.0)
    sums = jnp.dot(oht, x, preferred_element_type=jnp.float32)
    cnt = jnp.dot(oht, jnp.ones((N_NODES, 1), jnp.float32),
                  preferred_element_type=jnp.float32)
    pooled = sums / jnp.maximum(cnt, 1.0)
    y = jnp.dot(pooled, w1_ref[...], preferred_element_type=jnp.float32)
    y = y + b1_ref[...]
    y = jnp.dot(y, w3_ref[...], preferred_element_type=jnp.float32)
    y_ref[...] = y + b3_ref[...]


def kernel(x_s, x_t, edge_index, edge_attr, x_t_batch, x_s_batch,
           W_src, W_dst, att_src, att_dst, W_edge, att_edge, b_conv,
           W_fc1, b_fc1, W_fc3, b_fc3):
    f32 = jnp.float32
    w_src3 = W_src.reshape(D_FEAT, N_HEADS, N_CH)
    v_src = (w_src3 * att_src[None]).sum(-1)
    w_dst3 = W_dst.reshape(D_FEAT, N_HEADS, N_CH)
    v_dst = (w_dst3 * att_dst[None]).sum(-1)
    v_edge = (W_edge.reshape(4, N_HEADS, N_CH) * att_edge[None]).sum(-1)

    wcat = jnp.zeros((D_FEAT, ROW_W), f32)
    for h in range(N_HEADS):
        wcat = wcat.at[:, 16 * h:16 * h + N_CH].set(w_src3[:, h, :])
        wcat = wcat.at[:, 16 * h + 12].set(v_src[:, h])
    vdstp = jnp.zeros((D_FEAT, 8), f32).at[:, :3].set(v_dst)
    vedgep = jnp.zeros((4, 4), f32).at[:, :3].set(v_edge)
    vedge_blk = jnp.kron(jnp.eye(32, dtype=f32), vedgep)

    bpad = jnp.zeros((1, ROW_W), f32)
    for h in range(N_HEADS):
        bpad = bpad.at[0, 16 * h:16 * h + N_CH].set(
            b_conv[h * N_CH:(h + 1) * N_CH])
    w1p = jnp.zeros((ROW_W, 10), f32)
    for h in range(N_HEADS):
        w1p = w1p.at[16 * h:16 * h + N_CH, :].set(
            W_fc1[h * N_CH:(h + 1) * N_CH, :])

    ei = edge_index.astype(jnp.int32)
    sdpack = jnp.stack([ei[0].reshape(N_CHUNKS_TOT, K_EDGES),
                        ei[1].reshape(N_CHUNKS_TOT, K_EDGES)], axis=1)
    batch = x_s_batch.astype(jnp.int32).reshape(1, N_NODES)

    t_tab, adst_tab, ae_tab = pl.pallas_call(
        _prep_body,
        out_shape=(
            jax.ShapeDtypeStruct((N_NODES, ROW_W), f32),
            jax.ShapeDtypeStruct((N_NODES, 8), f32),
            jax.ShapeDtypeStruct((N_EDGES // 32, 128), f32),
        ),
    )(x_s, x_t, edge_attr.astype(f32).reshape(N_EDGES // 32, 128),
      wcat, vdstp, vedge_blk)
    ae_tab = ae_tab.reshape(N_EDGES, 4)

    part = _sc_edge(t_tab, adst_tab, sdpack, ae_tab)

    y = pl.pallas_call(
        _final_body,
        out_shape=jax.ShapeDtypeStruct((64, 1), f32),
    )(part, batch, bpad, w1p, b_fc1.reshape(1, 10), W_fc3,
      b_fc3.reshape(1, 1))
    return y.reshape(64)


# speedup vs baseline: 1.4737x; 1.4737x over previous
import functools

import jax
import jax.numpy as jnp
from jax import lax
from jax.experimental import pallas as pl
from jax.experimental.pallas import tpu as pltpu
from jax.experimental.pallas import tpu_sc as plsc

N_NODES = 10000
N_EDGES = 320000
D_FEAT = 128
N_HEADS = 3
N_CH = 12
ROW_W = 48
K_EDGES = 80
N_GRP = K_EDGES // 16
N_TILES = 16
N_WORKERS = 32
N_CHUNKS_TOT = N_EDGES // K_EDGES
CHUNKS_PER_W = N_CHUNKS_TOT // N_WORKERS
NBUF = 5
N_MACRO = CHUNKS_PER_W // NBUF
LAST = CHUNKS_PER_W - 1
ROWS_PER_TILE = 632


def _prep_body(xs_ref, xt_ref, wcat_ref, vdst_ref, t_ref, adst_ref):
    t_ref[...] = jnp.dot(xs_ref[...], wcat_ref[...],
                         preferred_element_type=jnp.float32)
    adst_ref[...] = jnp.dot(xt_ref[...], vdst_ref[...],
                            preferred_element_type=jnp.float32)


def _sc_edge_body(t_hbm, adst_hbm, sdp_hbm, attr_hbm, vedge_hbm,
                  part_hbm, *scr):
    sd = scr[0:5]
    aer = scr[5:10]
    rows = scr[10:15]
    adrow = scr[15:20]
    dscat = scr[20:25]
    vedge_v = scr[25]
    acc = scr[26]
    sem_i = scr[27:32]
    sem_g = scr[32:37]
    sem_s = scr[37:42]

    core = lax.axis_index("c")
    s = lax.axis_index("s")
    w = s * 2 + core

    iota16 = lax.iota(jnp.int32, 16)
    zero16 = jnp.zeros((16,), jnp.float32)

    def col_idx(col):
        return jnp.full((16,), col, jnp.int32)

    for r in range(K_EDGES):
        for o3 in range(ROW_W // 16):
            rows[0][r, pl.ds(o3 * 16, 16)] = zero16
    row_off = pl.multiple_of(
        jnp.minimum(s * ROWS_PER_TILE, N_NODES - ROWS_PER_TILE), 8)
    n_z = ROWS_PER_TILE // K_EDGES + 1
    for k in range(n_z):
        off_k = min(k * K_EDGES, ROWS_PER_TILE - K_EDGES)
        pltpu.sync_copy(rows[0],
                        acc.at[pl.ds(row_off + off_k, K_EDGES)])
    pltpu.sync_copy(vedge_hbm, vedge_v)
    vb = [[vedge_v[d * N_HEADS + h] for h in range(N_HEADS)]
          for d in range(4)]

    def issue_loads(c, b):
        chn = c * N_WORKERS + w
        pltpu.async_copy(sdp_hbm.at[chn], sd[b], sem_i[b])
        pltpu.async_copy(attr_hbm.at[pl.ds(chn * K_EDGES, K_EDGES)],
                         aer[b], sem_i[b])

    def wait_loads(c, b):
        chn = c * N_WORKERS + w
        pltpu.make_async_copy(sdp_hbm.at[chn], sd[b], sem_i[b]).wait()
        pltpu.make_async_copy(attr_hbm.at[pl.ds(chn * K_EDGES, K_EDGES)],
                              aer[b], sem_i[b]).wait()

    def issue_gather(b):
        pltpu.async_copy(t_hbm.at[sd[b].at[0]], rows[b], sem_g[b])
        pltpu.async_copy(adst_hbm.at[sd[b].at[1]], adrow[b], sem_g[b])

    def wait_gather(b):
        pltpu.make_async_copy(t_hbm.at[sd[b].at[0]], rows[b],
                              sem_g[b]).wait()
        pltpu.make_async_copy(adst_hbm.at[sd[b].at[1]], adrow[b],
                              sem_g[b]).wait()

    def issue_scatter(b):
        pltpu.async_copy(rows[b], acc.at[dscat[b]], sem_s[b], add=True)

    def wait_scatter(b):
        pltpu.make_async_copy(rows[b], acc.at[dscat[b]], sem_s[b]).wait()

    def compute(b):
        for g in range(N_GRP):
            r16 = iota16 + g * 16
            dst16 = sd[b][1, pl.ds(g * 16, 16)]
            dscat[b][pl.ds(g * 16, 16)] = dst16
            attr_d = [plsc.load_gather(aer[b], [r16, col_idx(d)])
                      for d in range(4)]
            for h in range(N_HEADS):
                a_src = plsc.load_gather(rows[b],
                                         [r16, col_idx(16 * h + 12)])
                a_dst = plsc.load_gather(adrow[b], [r16, col_idx(h)])
                a_e = (attr_d[0] * vb[0][h] + attr_d[1] * vb[1][h]
                       + attr_d[2] * vb[2][h] + attr_d[3] * vb[3][h])
                al = a_src + a_dst + a_e
                al = jnp.maximum(al, al * 0.2)
                ex = jnp.exp(al)
                plsc.store_scatter(rows[b], [r16, col_idx(16 * h + 12)], ex)
                for cc in range(N_CH):
                    col = 16 * h + cc
                    v = plsc.load_gather(rows[b], [r16, col_idx(col)])
                    plsc.store_scatter(rows[b], [r16, col_idx(col)], v * ex)

    plsc.subcore_barrier()

    for c in range(4):
        issue_loads(c, c)
    wait_loads(0, 0)
    wait_loads(1, 1)
    issue_gather(0)
    issue_gather(1)

    def macro_body(m, carry):
        for b in range(NBUF):
            c = m * NBUF + b
            wait_gather(b)

            @pl.when((c >= 3) & (c + 2 <= LAST))
            def _():
                wait_scatter((b + 2) % NBUF)

            @pl.when(c + 2 <= LAST)
            def _():
                wait_loads(c + 2, (b + 2) % NBUF)
                issue_gather((b + 2) % NBUF)

            @pl.when(c + 4 <= LAST)
            def _():
                issue_loads(c + 4, (b + 4) % NBUF)

            compute(b)
            issue_scatter(b)
        return carry

    lax.fori_loop(0, N_MACRO, macro_body, 0)
    for b in range(NBUF):
        wait_scatter(b)

    plsc.subcore_barrier()
    pltpu.sync_copy(acc.at[pl.ds(row_off, ROWS_PER_TILE)],
                    part_hbm.at[core, pl.ds(row_off, ROWS_PER_TILE)])


_sc_edge = functools.partial(
    pl.kernel,
    out_type=jax.ShapeDtypeStruct((2, N_NODES, ROW_W), jnp.float32),
    mesh=plsc.VectorSubcoreMesh(core_axis_name="c", subcore_axis_name="s"),
    compiler_params=pltpu.CompilerParams(needs_layout_passes=False,
                                         use_tc_tiling_on_sc=False),
    scratch_types=(
        [pltpu.VMEM((2, K_EDGES), jnp.int32) for _ in range(NBUF)]
        + [pltpu.VMEM((K_EDGES, 4), jnp.float32) for _ in range(NBUF)]
        + [pltpu.VMEM((K_EDGES, ROW_W), jnp.float32) for _ in range(NBUF)]
        + [pltpu.VMEM((K_EDGES, 8), jnp.float32) for _ in range(NBUF)]
        + [pltpu.VMEM((K_EDGES,), jnp.int32) for _ in range(NBUF)]
        + [pltpu.VMEM((16, 16), jnp.float32),
           pltpu.VMEM_SHARED((N_NODES, ROW_W), jnp.float32)]
        + [pltpu.SemaphoreType.DMA for _ in range(3 * NBUF)]
    ),
)(_sc_edge_body)


def _final_body(part_ref, batch_ref, bpad_ref, w1_ref, b1_ref,
                w3_ref, b3_ref, y_ref):
    p = part_ref[0] + part_ref[1]
    i2 = lax.broadcasted_iota(jnp.int32, (ROW_W, ROW_W), 0)
    j2 = lax.broadcasted_iota(jnp.int32, (ROW_W, ROW_W), 1)
    sel = jnp.where((i2 == (j2 // 16) * 16 + 12) & (j2 % 16 < 12), 1.0, 0.0)
    den = jnp.dot(p, sel, preferred_element_type=jnp.float32)
    x = jnp.maximum(p / (den + 1e-16) + bpad_ref[...], 0.0)
    bi = lax.broadcasted_iota(jnp.int32, (64, N_NODES), 0)
    oht = jnp.where(bi == batch_ref[...], 1.0, 0.0)
    sums = jnp.dot(oht, x, preferred_element_type=jnp.float32)
    cnt = jnp.dot(oht, jnp.ones((N_NODES, 1), jnp.float32),
                  preferred_element_type=jnp.float32)
    pooled = sums / jnp.maximum(cnt, 1.0)
    y = jnp.dot(pooled, w1_ref[...], preferred_element_type=jnp.float32)
    y = y + b1_ref[...]
    y = jnp.dot(y, w3_ref[...], preferred_element_type=jnp.float32)
    y_ref[...] = y + b3_ref[...]


def kernel(x_s, x_t, edge_index, edge_attr, x_t_batch, x_s_batch,
           W_src, W_dst, att_src, att_dst, W_edge, att_edge, b_conv,
           W_fc1, b_fc1, W_fc3, b_fc3):
    f32 = jnp.float32
    w_src3 = W_src.reshape(D_FEAT, N_HEADS, N_CH)
    v_src = (w_src3 * att_src[None]).sum(-1)
    w_dst3 = W_dst.reshape(D_FEAT, N_HEADS, N_CH)
    v_dst = (w_dst3 * att_dst[None]).sum(-1)
    v_edge = (W_edge.reshape(4, N_HEADS, N_CH) * att_edge[None]).sum(-1)

    wcat = jnp.zeros((D_FEAT, ROW_W), f32)
    for h in range(N_HEADS):
        wcat = wcat.at[:, 16 * h:16 * h + N_CH].set(w_src3[:, h, :])
        wcat = wcat.at[:, 16 * h + 12].set(v_src[:, h])
    vdstp = jnp.zeros((D_FEAT, 8), f32).at[:, :3].set(v_dst)
    vedge_bc = jnp.zeros((16, 16), f32)
    for d in range(4):
        for h in range(N_HEADS):
            vedge_bc = vedge_bc.at[d * N_HEADS + h, :].set(v_edge[d, h])

    bpad = jnp.zeros((1, ROW_W), f32)
    for h in range(N_HEADS):
        bpad = bpad.at[0, 16 * h:16 * h + N_CH].set(
            b_conv[h * N_CH:(h + 1) * N_CH])
    w1p = jnp.zeros((ROW_W, 10), f32)
    for h in range(N_HEADS):
        w1p = w1p.at[16 * h:16 * h + N_CH, :].set(
            W_fc1[h * N_CH:(h + 1) * N_CH, :])

    ei = edge_index.astype(jnp.int32)
    sdpack = jnp.stack([ei[0].reshape(N_CHUNKS_TOT, K_EDGES),
                        ei[1].reshape(N_CHUNKS_TOT, K_EDGES)], axis=1)
    batch = x_s_batch.astype(jnp.int32).reshape(1, N_NODES)

    t_tab, adst_tab = pl.pallas_call(
        _prep_body,
        out_shape=(
            jax.ShapeDtypeStruct((N_NODES, ROW_W), f32),
            jax.ShapeDtypeStruct((N_NODES, 8), f32),
        ),
    )(x_s, x_t, wcat, vdstp)

    part = _sc_edge(t_tab, adst_tab, sdpack, edge_attr.astype(f32),
                    vedge_bc)

    y = pl.pallas_call(
        _final_body,
        out_shape=jax.ShapeDtypeStruct((64, 1), f32),
    )(part, batch, bpad, w1p, b_fc1.reshape(1, 10), W_fc3,
      b_fc3.reshape(1, 1))
    return y.reshape(64)


# speedup vs baseline: 1.4879x; 1.0097x over previous
import functools

import jax
import jax.numpy as jnp
from jax import lax
from jax.experimental import pallas as pl
from jax.experimental.pallas import tpu as pltpu
from jax.experimental.pallas import tpu_sc as plsc

N_NODES = 10000
N_EDGES = 320000
D_FEAT = 128
N_HEADS = 3
N_CH = 12
ROW_W = 48
K_EDGES = 80
N_GRP = K_EDGES // 16
N_TILES = 16
N_WORKERS = 32
N_CHUNKS_TOT = N_EDGES // K_EDGES
CHUNKS_PER_W = N_CHUNKS_TOT // N_WORKERS
NBUF = 5
N_MACRO = CHUNKS_PER_W // NBUF
LAST = CHUNKS_PER_W - 1
ROWS_PER_TILE = 632


def _prep_body(xs_ref, xt_ref, wcat_ref, vdst_ref, t_ref, adst_ref):
    t_ref[...] = jnp.dot(xs_ref[...], wcat_ref[...],
                         preferred_element_type=jnp.float32)
    adst_ref[...] = jnp.dot(xt_ref[...], vdst_ref[...],
                            preferred_element_type=jnp.float32)


def _sc_edge_body(t_hbm, adst_hbm, sdp_hbm, attr_hbm, vedge_hbm,
                  part_hbm, *scr):
    sd = scr[0:5]
    aer = scr[5:10]
    rows = scr[10:15]
    adrow = scr[15:20]
    dscat = scr[20:25]
    vedge_v = scr[25]
    acc = scr[26]
    sem_i = scr[27:32]
    sem_g = scr[32:37]
    sem_s = scr[37:42]

    core = lax.axis_index("c")
    s = lax.axis_index("s")
    w = s * 2 + core

    iota16 = lax.iota(jnp.int32, 16)
    zero16 = jnp.zeros((16,), jnp.float32)

    def col_idx(col):
        return jnp.full((16,), col, jnp.int32)

    for r in range(K_EDGES):
        for o3 in range(ROW_W // 16):
            rows[0][r, pl.ds(o3 * 16, 16)] = zero16
    row_off = pl.multiple_of(
        jnp.minimum(s * ROWS_PER_TILE, N_NODES - ROWS_PER_TILE), 8)
    n_z = ROWS_PER_TILE // K_EDGES + 1
    for k in range(n_z):
        off_k = min(k * K_EDGES, ROWS_PER_TILE - K_EDGES)
        pltpu.sync_copy(rows[0],
                        acc.at[pl.ds(row_off + off_k, K_EDGES)])
    pltpu.sync_copy(vedge_hbm, vedge_v)
    vb = [[vedge_v[d * N_HEADS + h] for h in range(N_HEADS)]
          for d in range(4)]

    def issue_loads(c, b):
        chn = c * N_WORKERS + w
        pltpu.async_copy(sdp_hbm.at[chn], sd[b], sem_i[b])
        pltpu.async_copy(attr_hbm.at[pl.ds(chn * K_EDGES, K_EDGES)],
                         aer[b], sem_i[b])

    def wait_loads(c, b):
        chn = c * N_WORKERS + w
        pltpu.make_async_copy(sdp_hbm.at[chn], sd[b], sem_i[b]).wait()
        pltpu.make_async_copy(attr_hbm.at[pl.ds(chn * K_EDGES, K_EDGES)],
                              aer[b], sem_i[b]).wait()

    def issue_gather(b):
        pltpu.async_copy(t_hbm.at[sd[b].at[0]], rows[b], sem_g[b])
        pltpu.async_copy(adst_hbm.at[sd[b].at[1]], adrow[b], sem_g[b])

    def wait_gather(b):
        pltpu.make_async_copy(t_hbm.at[sd[b].at[0]], rows[b],
                              sem_g[b]).wait()
        pltpu.make_async_copy(adst_hbm.at[sd[b].at[1]], adrow[b],
                              sem_g[b]).wait()

    def issue_scatter(b):
        pltpu.async_copy(rows[b], acc.at[dscat[b]], sem_s[b], add=True)

    def wait_scatter(b):
        pltpu.make_async_copy(rows[b], acc.at[dscat[b]], sem_s[b]).wait()

    def compute(b):
        for g in range(N_GRP):
            r16 = iota16 + g * 16
            dst16 = sd[b][1, pl.ds(g * 16, 16)]
            dscat[b][pl.ds(g * 16, 16)] = dst16
            attr_d = [plsc.load_gather(aer[b], [r16, col_idx(d)])
                      for d in range(4)]
            for h in range(N_HEADS):
                a_src = plsc.load_gather(rows[b],
                                         [r16, col_idx(16 * h + 12)])
                a_dst = plsc.load_gather(adrow[b], [r16, col_idx(h)])
                a_e = (attr_d[0] * vb[0][h] + attr_d[1] * vb[1][h]
                       + attr_d[2] * vb[2][h] + attr_d[3] * vb[3][h])
                al = a_src + a_dst + a_e
                al = jnp.maximum(al, al * 0.2)
                ex = jnp.exp(al)
                plsc.store_scatter(rows[b], [r16, col_idx(16 * h + 12)], ex)
                for cc in range(N_CH):
                    col = 16 * h + cc
                    v = plsc.load_gather(rows[b], [r16, col_idx(col)])
                    plsc.store_scatter(rows[b], [r16, col_idx(col)], v * ex)

    plsc.subcore_barrier()

    for c in range(NBUF):
        issue_loads(c, c)
    for c in range(3):
        wait_loads(c, c)
        issue_gather(c)

    def macro_body(m, carry):
        for b in range(NBUF):
            c = m * NBUF + b
            wait_gather(b)

            @pl.when((c >= 2) & (c + 3 <= LAST))
            def _():
                wait_scatter((b + 3) % NBUF)

            @pl.when(c + 3 <= LAST)
            def _():
                wait_loads(c + 3, (b + 3) % NBUF)
                issue_gather((b + 3) % NBUF)

            compute(b)

            @pl.when(c + 5 <= LAST)
            def _():
                issue_loads(c + 5, b)

            issue_scatter(b)
        return carry

    lax.fori_loop(0, N_MACRO, macro_body, 0)
    for b in range(NBUF):
        wait_scatter(b)

    plsc.subcore_barrier()
    pltpu.sync_copy(acc.at[pl.ds(row_off, ROWS_PER_TILE)],
                    part_hbm.at[core, pl.ds(row_off, ROWS_PER_TILE)])


_sc_edge = functools.partial(
    pl.kernel,
    out_type=jax.ShapeDtypeStruct((2, N_NODES, ROW_W), jnp.float32),
    mesh=plsc.VectorSubcoreMesh(core_axis_name="c", subcore_axis_name="s"),
    compiler_params=pltpu.CompilerParams(needs_layout_passes=False,
                                         use_tc_tiling_on_sc=False),
    scratch_types=(
        [pltpu.VMEM((2, K_EDGES), jnp.int32) for _ in range(NBUF)]
        + [pltpu.VMEM((K_EDGES, 4), jnp.float32) for _ in range(NBUF)]
        + [pltpu.VMEM((K_EDGES, ROW_W), jnp.float32) for _ in range(NBUF)]
        + [pltpu.VMEM((K_EDGES, 8), jnp.float32) for _ in range(NBUF)]
        + [pltpu.VMEM((K_EDGES,), jnp.int32) for _ in range(NBUF)]
        + [pltpu.VMEM((16, 16), jnp.float32),
           pltpu.VMEM_SHARED((N_NODES, ROW_W), jnp.float32)]
        + [pltpu.SemaphoreType.DMA for _ in range(3 * NBUF)]
    ),
)(_sc_edge_body)


def _final_body(part_ref, batch_ref, bpad_ref, w1_ref, b1_ref,
                w3_ref, b3_ref, y_ref):
    p = part_ref[0] + part_ref[1]
    i2 = lax.broadcasted_iota(jnp.int32, (ROW_W, ROW_W), 0)
    j2 = lax.broadcasted_iota(jnp.int32, (ROW_W, ROW_W), 1)
    sel = jnp.where((i2 == (j2 // 16) * 16 + 12) & (j2 % 16 < 12), 1.0, 0.0)
    den = jnp.dot(p, sel, preferred_element_type=jnp.float32)
    x = jnp.maximum(p / (den + 1e-16) + bpad_ref[...], 0.0)
    bi = lax.broadcasted_iota(jnp.int32, (64, N_NODES), 0)
    oht = jnp.where(bi == batch_ref[...], 1.0, 0.0)
    sums = jnp.dot(oht, x, preferred_element_type=jnp.float32)
    cnt = jnp.dot(oht, jnp.ones((N_NODES, 1), jnp.float32),
                  preferred_element_type=jnp.float32)
    pooled = sums / jnp.maximum(cnt, 1.0)
    y = jnp.dot(pooled, w1_ref[...], preferred_element_type=jnp.float32)
    y = y + b1_ref[...]
    y = jnp.dot(y, w3_ref[...], preferred_element_type=jnp.float32)
    y_ref[...] = y + b3_ref[...]


def kernel(x_s, x_t, edge_index, edge_attr, x_t_batch, x_s_batch,
           W_src, W_dst, att_src, att_dst, W_edge, att_edge, b_conv,
           W_fc1, b_fc1, W_fc3, b_fc3):
    f32 = jnp.float32
    w_src3 = W_src.reshape(D_FEAT, N_HEADS, N_CH)
    v_src = (w_src3 * att_src[None]).sum(-1)
    w_dst3 = W_dst.reshape(D_FEAT, N_HEADS, N_CH)
    v_dst = (w_dst3 * att_dst[None]).sum(-1)
    v_edge = (W_edge.reshape(4, N_HEADS, N_CH) * att_edge[None]).sum(-1)

    wcat = jnp.zeros((D_FEAT, ROW_W), f32)
    for h in range(N_HEADS):
        wcat = wcat.at[:, 16 * h:16 * h + N_CH].set(w_src3[:, h, :])
        wcat = wcat.at[:, 16 * h + 12].set(v_src[:, h])
    vdstp = jnp.zeros((D_FEAT, 8), f32).at[:, :3].set(v_dst)
    vedge_bc = jnp.zeros((16, 16), f32)
    for d in range(4):
        for h in range(N_HEADS):
            vedge_bc = vedge_bc.at[d * N_HEADS + h, :].set(v_edge[d, h])

    bpad = jnp.zeros((1, ROW_W), f32)
    for h in range(N_HEADS):
        bpad = bpad.at[0, 16 * h:16 * h + N_CH].set(
            b_conv[h * N_CH:(h + 1) * N_CH])
    w1p = jnp.zeros((ROW_W, 10), f32)
    for h in range(N_HEADS):
        w1p = w1p.at[16 * h:16 * h + N_CH, :].set(
            W_fc1[h * N_CH:(h + 1) * N_CH, :])

    ei = edge_index.astype(jnp.int32)
    sdpack = jnp.stack([ei[0].reshape(N_CHUNKS_TOT, K_EDGES),
                        ei[1].reshape(N_CHUNKS_TOT, K_EDGES)], axis=1)
    batch = x_s_batch.astype(jnp.int32).reshape(1, N_NODES)

    t_tab, adst_tab = pl.pallas_call(
        _prep_body,
        out_shape=(
            jax.ShapeDtypeStruct((N_NODES, ROW_W), f32),
            jax.ShapeDtypeStruct((N_NODES, 8), f32),
        ),
    )(x_s, x_t, wcat, vdstp)

    part = _sc_edge(t_tab, adst_tab, sdpack, edge_attr.astype(f32),
                    vedge_bc)

    y = pl.pallas_call(
        _final_body,
        out_shape=jax.ShapeDtypeStruct((64, 1), f32),
    )(part, batch, bpad, w1p, b_fc1.reshape(1, 10), W_fc3,
      b_fc3.reshape(1, 1))
    return y.reshape(64)
